# Initial kernel scaffold; baseline (speedup 1.0000x reference)
import jax
import jax.numpy as jnp
from jax.experimental import pallas as pl


def kernel(preds):
    raise NotImplementedError("write your pallas kernel here")


# speedup vs baseline: 1.3195x; 1.3195x over previous
import functools

import jax
import jax.numpy as jnp
from jax.experimental import pallas as pl
from jax.experimental.pallas import tpu as pltpu

_H = 64
_W = 64
_HW = _H * _W
_NO = 85
_AW = (10.0, 16.0, 33.0)
_AH = (13.0, 30.0, 23.0)


def _decode_block(v, a):
    e = jnp.exp(-jnp.abs(v))
    d = 1.0 / (1.0 + e)
    sig = jnp.where(v >= 0, d, e * d)
    ex = jnp.exp(v)
    ch = jax.lax.broadcasted_iota(jnp.int32, v.shape, 0)
    pos = jax.lax.broadcasted_iota(jnp.int32, v.shape, 1)
    y = (pos // _W).astype(jnp.float32)
    x = (pos % _W).astype(jnp.float32)
    aw = jnp.where(a == 0, _AW[0], jnp.where(a == 1, _AW[1], _AW[2]))
    ah = jnp.where(a == 0, _AH[0], jnp.where(a == 1, _AH[1], _AH[2]))
    return jnp.where(
        ch == 0, (sig + y) * 8.0,
        jnp.where(ch == 1, (sig + x) * 8.0,
                  jnp.where(ch == 2, ex * aw,
                            jnp.where(ch == 3, ex * ah, sig))))


def _tc_body(x_ref, o_ref):
    a = pl.program_id(0) % 3
    r = _decode_block(x_ref[0], a)
    o_ref[0] = r.T


@functools.partial(jax.jit, static_argnames=("interpret",))
def kernel(preds, interpret=False):
    n = preds.shape[0]
    x = preds.reshape(n * 3, _NO, _HW)
    out = pl.pallas_call(
        _tc_body,
        grid=(n * 3,),
        in_specs=[pl.BlockSpec((1, _NO, _HW), lambda i: (i, 0, 0))],
        out_specs=pl.BlockSpec((1, _HW, _NO), lambda i: (i, 0, 0)),
        out_shape=jax.ShapeDtypeStruct((n * 3, _HW, _NO), jnp.float32),
        interpret=interpret,
    )(x)
    return out.reshape(n, 3 * _HW, _NO)
